# baseline (speedup 1.0000x reference)
import jax
import jax.numpy as jnp
from jax import lax
from jax.experimental import pallas as pl
from jax.experimental.pallas import tpu as pltpu
from jax.experimental.pallas import tpu_sc as plsc

B, L, D = 4, 8192, 1024
N_TOK = B * L
NC, NS = 2, 16
NW = NC * NS
TOK_PER_W = N_TOK // NW
ROWS_PER_BLK = 16
N_SLOTS = 4
SS_ROWS = N_SLOTS * ROWS_PER_BLK
N_SS = TOK_PER_W // SS_ROWS
N_GRP = D // 16

def _sc_body(table_hbm, idx_hbm, out_hbm, idx_v, t0_v, t1_v, ring, sems):
    wid = lax.axis_index("s") * NC + lax.axis_index("c")
    base = wid * TOK_PER_W
    pltpu.sync_copy(table_hbm.at[0], t0_v)
    pltpu.sync_copy(table_hbm.at[1], t1_v)
    pltpu.sync_copy(idx_hbm.at[pl.ds(base, TOK_PER_W)], idx_v)

    def superstep(ss, carry):
        for slot in range(N_SLOTS):
            slot_row = slot * ROWS_PER_BLK
            rowbase = ss * SS_ROWS + slot_row

            @pl.when(ss > 0)
            def _drain():
                pltpu.make_async_copy(
                    ring.at[pl.ds(slot_row, ROWS_PER_BLK)],
                    out_hbm.at[pl.ds(base + rowbase, ROWS_PER_BLK)],
                    sems[slot],
                ).wait()

            idvi = idx_v[pl.ds(rowbase, 16)]
            row_masks = [
                -idvi.at[jnp.full((16,), j, dtype=jnp.int32)].get(
                    mode="promise_in_bounds"
                )
                for j in range(16)
            ]

            def col_group(g, c):
                x0 = lax.bitcast_convert_type(t0_v[pl.ds(g * 16, 16)], jnp.int32)
                x1 = lax.bitcast_convert_type(t1_v[pl.ds(g * 16, 16)], jnp.int32)
                xd = x0 ^ x1
                for j in range(ROWS_PER_BLK):
                    val = x0 ^ (xd & row_masks[j])
                    ring[slot_row + j, pl.ds(g * 16, 16)] = (
                        lax.bitcast_convert_type(val, jnp.float32)
                    )
                return c

            lax.fori_loop(0, N_GRP, col_group, 0, unroll=4)
            pltpu.async_copy(
                ring.at[pl.ds(slot_row, ROWS_PER_BLK)],
                out_hbm.at[pl.ds(base + rowbase, ROWS_PER_BLK)],
                sems[slot],
            )
        return carry

    lax.fori_loop(0, N_SS, superstep, 0)
    for slot in range(N_SLOTS):
        slot_row = slot * ROWS_PER_BLK
        last = (N_SS - 1) * SS_ROWS + slot_row
        pltpu.make_async_copy(
            ring.at[pl.ds(slot_row, ROWS_PER_BLK)],
            out_hbm.at[pl.ds(base + last, ROWS_PER_BLK)],
            sems[slot],
        ).wait()


@jax.jit
def _lookup(ids_flat, table):
    mesh = plsc.VectorSubcoreMesh(core_axis_name="c", subcore_axis_name="s")
    run = pl.kernel(
        _sc_body,
        out_type=jax.ShapeDtypeStruct((N_TOK, D), jnp.float32),
        mesh=mesh,
        scratch_types=[
            pltpu.VMEM((TOK_PER_W,), jnp.int32),
            pltpu.VMEM((D,), jnp.float32),
            pltpu.VMEM((D,), jnp.float32),
            pltpu.VMEM((SS_ROWS, D), jnp.float32),
            [pltpu.SemaphoreType.DMA for _ in range(N_SLOTS)],
        ],
    )
    return run(table, ids_flat)


def kernel(token_type_ids, table):
    ids_flat = token_type_ids.reshape(-1).astype(jnp.int32)
    out = _lookup(ids_flat, table)
    return out.reshape(token_type_ids.shape + (D,))


# speedup vs baseline: 1.0328x; 1.0328x over previous
import jax
import jax.numpy as jnp
from jax import lax
from jax.experimental import pallas as pl
from jax.experimental.pallas import tpu as pltpu
from jax.experimental.pallas import tpu_sc as plsc

B, L, D = 4, 8192, 1024
N_TOK = B * L
NC, NS = 2, 16
NW = NC * NS
TOK_PER_W = N_TOK // NW
ROWS_PER_BLK = 16
N_SLOTS = 4
SS_ROWS = N_SLOTS * ROWS_PER_BLK
N_SS = TOK_PER_W // SS_ROWS
N_GRP = D // 16

def _sc_body(table_hbm, idx_hbm, out_hbm, idx_v, t0_v, t1_v, ring, sems):
    wid = lax.axis_index("s") * NC + lax.axis_index("c")
    base = wid * TOK_PER_W
    pltpu.sync_copy(table_hbm.at[0], t0_v)
    pltpu.sync_copy(table_hbm.at[1], t1_v)
    pltpu.sync_copy(idx_hbm.at[pl.ds(base, TOK_PER_W)], idx_v)

    def superstep(ss, carry):
        for slot in range(N_SLOTS):
            slot_row = slot * ROWS_PER_BLK
            rowbase = ss * SS_ROWS + slot_row

            @pl.when(ss > 0)
            def _drain():
                pltpu.make_async_copy(
                    ring.at[pl.ds(slot_row, ROWS_PER_BLK)],
                    out_hbm.at[pl.ds(base + rowbase, ROWS_PER_BLK)],
                    sems[slot],
                ).wait()

            idvi = idx_v[pl.ds(rowbase, 16)]
            row_masks = [
                -idvi.at[jnp.full((16,), j, dtype=jnp.int32)].get(
                    mode="promise_in_bounds"
                )
                for j in range(16)
            ]

            def col_group(g, c):
                x0 = lax.bitcast_convert_type(t0_v[pl.ds(g * 16, 16)], jnp.int32)
                x1 = lax.bitcast_convert_type(t1_v[pl.ds(g * 16, 16)], jnp.int32)
                xd = x0 ^ x1
                for j in range(ROWS_PER_BLK):
                    val = x0 ^ (xd & row_masks[j])
                    ring[slot_row + j, pl.ds(g * 16, 16)] = (
                        lax.bitcast_convert_type(val, jnp.float32)
                    )
                return c

            lax.fori_loop(0, N_GRP, col_group, 0, unroll=2)
            pltpu.async_copy(
                ring.at[pl.ds(slot_row, ROWS_PER_BLK)],
                out_hbm.at[pl.ds(base + rowbase, ROWS_PER_BLK)],
                sems[slot],
            )
        return carry

    lax.fori_loop(0, N_SS, superstep, 0)
    for slot in range(N_SLOTS):
        slot_row = slot * ROWS_PER_BLK
        last = (N_SS - 1) * SS_ROWS + slot_row
        pltpu.make_async_copy(
            ring.at[pl.ds(slot_row, ROWS_PER_BLK)],
            out_hbm.at[pl.ds(base + last, ROWS_PER_BLK)],
            sems[slot],
        ).wait()


@jax.jit
def _lookup(ids_flat, table):
    mesh = plsc.VectorSubcoreMesh(core_axis_name="c", subcore_axis_name="s")
    run = pl.kernel(
        _sc_body,
        out_type=jax.ShapeDtypeStruct((N_TOK, D), jnp.float32),
        mesh=mesh,
        scratch_types=[
            pltpu.VMEM((TOK_PER_W,), jnp.int32),
            pltpu.VMEM((D,), jnp.float32),
            pltpu.VMEM((D,), jnp.float32),
            pltpu.VMEM((SS_ROWS, D), jnp.float32),
            [pltpu.SemaphoreType.DMA for _ in range(N_SLOTS)],
        ],
    )
    return run(table, ids_flat)


def kernel(token_type_ids, table):
    ids_flat = token_type_ids.reshape(-1).astype(jnp.int32)
    out = _lookup(ids_flat, table)
    return out.reshape(token_type_ids.shape + (D,))
